# baseline (device time: 51746 ns/iter reference)
import jax
import jax.numpy as jnp
from jax import lax
from jax.experimental import pallas as pl
from jax.experimental.pallas import tpu as pltpu

N_DEV = 4


def kernel(t, W):
    m_per, k = t.shape
    _, n = W.shape
    mh = m_per // 2
    mq = m_per // 4
    me = m_per // 8

    def body(t_ref, w_ref, out_ref, c1s, rs1r, rs2s, rs2r, yb, wb, tfq, tqb,
             sems_s, sems_r):
        my = lax.axis_index("i")
        p_a = my ^ 1
        p_b = 3 - my
        a_bit = my & 1
        b_bit = my // 2
        keep1 = a_bit ^ b_bit
        keep2 = b_bit
        q1 = b_bit
        q2 = a_bit
        row1 = keep1 * mq + q1 * me
        l2 = keep2 * mq + q2 * me
        row2 = mh + l2

        barrier_sem = pltpu.get_barrier_semaphore()
        for nbr in (p_a, p_b):
            pl.semaphore_signal(
                barrier_sem, inc=1,
                device_id=(nbr,), device_id_type=pl.DeviceIdType.MESH,
            )
        c1s[0, :, :] = t_ref[pl.ds((1 - keep1) * mq, mq), :].astype(jnp.bfloat16)
        c1s[1, :, :] = t_ref[pl.ds(mh + (1 - keep2) * mq, mq), :].astype(
            jnp.bfloat16
        )
        pl.semaphore_wait(barrier_sem, 2)

        def xchg(sem_idx, src, dst, target):
            return pltpu.make_async_remote_copy(
                src_ref=src, dst_ref=dst,
                send_sem=sems_s.at[sem_idx], recv_sem=sems_r.at[sem_idx],
                device_id=(target,), device_id_type=pl.DeviceIdType.MESH,
            )

        r1a = xchg(0, c1s.at[0, pl.ds((1 - q1) * me, me), :],
                   rs1r.at[0, pl.ds((1 - q1) * me, me), :], p_a)
        r1a.start()
        r2a = xchg(1, c1s.at[1, pl.ds(q2 * me, me), :],
                   rs1r.at[1, pl.ds(q2 * me, me), :], p_b)
        r2a.start()
        r1b = xchg(10, c1s.at[0, pl.ds(q1 * me, me), :],
                   rs1r.at[0, pl.ds(q1 * me, me), :], p_a)
        r1b.start()
        r2b = xchg(11, c1s.at[1, pl.ds((1 - q2) * me, me), :],
                   rs1r.at[1, pl.ds((1 - q2) * me, me), :], p_b)
        r2b.start()

        tfq[0, :, :] = t_ref[pl.ds(keep1 * mq + (1 - q1) * me, me), :].astype(
            jnp.bfloat16
        )
        tfq[1, :, :] = t_ref[
            pl.ds(mh + keep2 * mq + (1 - q2) * me, me), :
        ].astype(jnp.bfloat16)
        tqb[0, :, :] = t_ref[pl.ds(row1, me), :].astype(jnp.bfloat16)
        tqb[1, :, :] = t_ref[pl.ds(row2, me), :].astype(jnp.bfloat16)
        wb[:, :] = w_ref[:, :].astype(jnp.bfloat16)

        r1a.wait()
        rs2s[0, :, :] = rs1r[0, pl.ds((1 - q1) * me, me), :] + tfq[0, :, :]
        r3 = xchg(2, rs2s.at[0], rs2r.at[0], p_b)
        r3.start()

        r2a.wait()
        rs2s[1, :, :] = rs1r[1, pl.ds((1 - q2) * me, me), :] + tfq[1, :, :]
        r4 = xchg(3, rs2s.at[1], rs2r.at[1], p_a)
        r4.start()

        r1b.wait()
        r3.wait()
        s1 = rs2r[0, :, :] + rs1r[0, pl.ds(q1 * me, me), :] + tqb[0, :, :]
        y1 = lax.dot_general(
            s1, wb[:, :],
            dimension_numbers=(((1,), (0,)), ((), ())),
            preferred_element_type=jnp.float32,
        )
        yb[0, pl.ds(row1, me), :] = y1.astype(jnp.bfloat16)
        g1 = xchg(4, yb.at[0, pl.ds(row1, me), :],
                  yb.at[0, pl.ds(row1, me), :], p_b)
        g1.start()
        g3m = xchg(6, yb.at[0, pl.ds(row1, me), :],
                   yb.at[0, pl.ds(row1, me), :], p_a)
        g3m.start()
        out_ref[pl.ds(row1, me), :] = y1

        r2b.wait()
        r4.wait()
        s2 = rs2r[1, :, :] + rs1r[1, pl.ds(q2 * me, me), :] + tqb[1, :, :]
        y2 = lax.dot_general(
            s2, wb[:, :],
            dimension_numbers=(((1,), (0,)), ((), ())),
            preferred_element_type=jnp.float32,
        )
        yb[1, pl.ds(l2, me), :] = y2.astype(jnp.bfloat16)
        g2 = xchg(5, yb.at[1, pl.ds(l2, me), :],
                  yb.at[1, pl.ds(l2, me), :], p_a)
        g2.start()
        g4m = xchg(8, yb.at[1, pl.ds(l2, me), :],
                   yb.at[1, pl.ds(l2, me), :], p_b)
        g4m.start()
        out_ref[pl.ds(row2, me), :] = y2

        pq1 = keep1 * mq + (1 - q1) * me
        g1.wait()
        g3p = xchg(7, yb.at[0, pl.ds(pq1, me), :],
                   yb.at[0, pl.ds(pq1, me), :], p_a)
        g3p.start()
        out_ref[pl.ds(pq1, me), :] = yb[0, pl.ds(pq1, me), :].astype(jnp.float32)

        pq2 = keep2 * mq + (1 - q2) * me
        g2.wait()
        g4p = xchg(9, yb.at[1, pl.ds(pq2, me), :],
                   yb.at[1, pl.ds(pq2, me), :], p_b)
        g4p.start()
        out_ref[pl.ds(mh + pq2, me), :] = yb[1, pl.ds(pq2, me), :].astype(
            jnp.float32
        )

        oq1a = (1 - keep1) * mq + q1 * me
        oq1b = (1 - keep1) * mq + (1 - q1) * me
        g3m.wait()
        out_ref[pl.ds(oq1a, me), :] = yb[0, pl.ds(oq1a, me), :].astype(
            jnp.float32
        )
        oq2a = (1 - keep2) * mq + (1 - q2) * me
        oq2b = (1 - keep2) * mq + q2 * me
        g4m.wait()
        out_ref[pl.ds(mh + oq2a, me), :] = yb[1, pl.ds(oq2a, me), :].astype(
            jnp.float32
        )
        g3p.wait()
        out_ref[pl.ds(oq1b, me), :] = yb[0, pl.ds(oq1b, me), :].astype(
            jnp.float32
        )
        g4p.wait()
        out_ref[pl.ds(mh + oq2b, me), :] = yb[1, pl.ds(oq2b, me), :].astype(
            jnp.float32
        )

    return pl.pallas_call(
        body,
        out_shape=jax.ShapeDtypeStruct((m_per, n), jnp.float32),
        in_specs=[
            pl.BlockSpec(memory_space=pltpu.VMEM),
            pl.BlockSpec(memory_space=pltpu.VMEM),
        ],
        out_specs=pl.BlockSpec(memory_space=pltpu.VMEM),
        scratch_shapes=[
            pltpu.VMEM((2, mq, k), jnp.bfloat16),
            pltpu.VMEM((2, mq, k), jnp.bfloat16),
            pltpu.VMEM((2, me, k), jnp.bfloat16),
            pltpu.VMEM((2, me, k), jnp.bfloat16),
            pltpu.VMEM((2, mh, n), jnp.bfloat16),
            pltpu.VMEM((k, n), jnp.bfloat16),
            pltpu.VMEM((2, m_per // 8, k), jnp.bfloat16),
            pltpu.VMEM((2, m_per // 8, k), jnp.bfloat16),
            pltpu.SemaphoreType.DMA((12,)),
            pltpu.SemaphoreType.DMA((12,)),
        ],
        compiler_params=pltpu.CompilerParams(collective_id=0),
    )(t, W)
